# baseline (speedup 1.0000x reference)
import functools

import jax
import jax.numpy as jnp
from jax import lax
from jax.experimental import pallas as pl
from jax.experimental.pallas import tpu as pltpu
from jax.experimental.pallas import tpu_sc as plsc

_NC = 2
_NS = 16
_NW = _NC * _NS
_L = 16

_N = 16384 * 100
_MINOR = 128
_ROWS = _N // _MINOR
_ROWS_W = _ROWS // _NW
_VPR = _MINOR // _L


def _sc_body(x_hbm, w_hbm, out_hbm, x_v, o_v, w_v):
    wid = lax.axis_index("s") * _NC + lax.axis_index("c")
    row0 = wid * _ROWS_W

    pltpu.sync_copy(x_hbm.at[pl.ds(row0, _ROWS_W), :], x_v)
    pltpu.sync_copy(w_hbm, w_v)
    w = w_v[...]
    e = jnp.exp(w + w)
    lut = (e - 1.0) / (e + 1.0)

    def row_step(r, carry):
        for v in range(_VPR):
            idx = x_v[r, pl.ds(v * _L, _L)]
            o_v[r, pl.ds(v * _L, _L)] = lut.at[idx].get(mode="promise_in_bounds")
        return carry

    lax.fori_loop(0, _ROWS_W, row_step, 0, unroll=2)

    pltpu.sync_copy(o_v, out_hbm.at[pl.ds(row0, _ROWS_W), :])


@functools.partial(jax.jit, static_argnames=())
def _run(x, w_pad):
    mesh = plsc.VectorSubcoreMesh(core_axis_name="c", subcore_axis_name="s")
    f = pl.kernel(
        _sc_body,
        out_type=jax.ShapeDtypeStruct((_ROWS, _MINOR), jnp.float32),
        mesh=mesh,
        scratch_types=[
            pltpu.VMEM((_ROWS_W, _MINOR), jnp.int32),
            pltpu.VMEM((_ROWS_W, _MINOR), jnp.float32),
            pltpu.VMEM((_L,), jnp.float32),
        ],
    )
    return f(x, w_pad)


def kernel(x, data_bias_weight):
    xr = x.reshape(_ROWS, _MINOR)
    w_pad = jnp.pad(data_bias_weight.reshape(3), (0, _L - 3))
    return _run(xr, w_pad).reshape(16384, 100, 1)


# speedup vs baseline: 2.4224x; 2.4224x over previous
import functools

import jax
import jax.numpy as jnp
from jax import lax
from jax.experimental import pallas as pl
from jax.experimental.pallas import tpu as pltpu
from jax.experimental.pallas import tpu_sc as plsc

_NC = 2
_NS = 16
_NW = _NC * _NS
_L = 16

_F = 100
_B = 16384
_COLS_W = _B // _NW
_VPR = _COLS_W // _L


def _sc_body(x_hbm, w_hbm, out_hbm, x_v, o_v, w_v):
    wid = lax.axis_index("s") * _NC + lax.axis_index("c")
    c0 = wid * _COLS_W

    pltpu.sync_copy(x_hbm.at[:, pl.ds(c0, _COLS_W)], x_v)
    pltpu.sync_copy(w_hbm, w_v)
    w = w_v[...]
    e = jnp.exp(w + w)
    lut = (e - 1.0) / (e + 1.0)

    def row_step(r, carry):
        for v in range(_VPR):
            idx = x_v[r, pl.ds(v * _L, _L)]
            o_v[r, pl.ds(v * _L, _L)] = lut.at[idx].get(mode="promise_in_bounds")
        return carry

    lax.fori_loop(0, _F, row_step, 0)

    pltpu.sync_copy(o_v, out_hbm.at[:, pl.ds(c0, _COLS_W)])


@functools.partial(jax.jit, static_argnames=())
def _run(xt, w_pad):
    mesh = plsc.VectorSubcoreMesh(core_axis_name="c", subcore_axis_name="s")
    f = pl.kernel(
        _sc_body,
        out_type=jax.ShapeDtypeStruct((_F, _B), jnp.float32),
        mesh=mesh,
        scratch_types=[
            pltpu.VMEM((_F, _COLS_W), jnp.int32),
            pltpu.VMEM((_F, _COLS_W), jnp.float32),
            pltpu.VMEM((_L,), jnp.float32),
        ],
    )
    return f(xt, w_pad)


def kernel(x, data_bias_weight):
    xt = x.T
    w_pad = jnp.pad(data_bias_weight.reshape(3), (0, _L - 3))
    yt = _run(xt, w_pad)
    return yt.T.reshape(_B, _F, 1)


# speedup vs baseline: 2.4509x; 1.0118x over previous
import functools

import jax
import jax.numpy as jnp
from jax import lax
from jax.experimental import pallas as pl
from jax.experimental.pallas import tpu as pltpu
from jax.experimental.pallas import tpu_sc as plsc

_NC = 2
_NS = 16
_NW = _NC * _NS
_L = 16

_F = 100
_B = 16384
_COLS_W = _B // _NW
_CHUNK = 128
_NCH = _COLS_W // _CHUNK
_VPC = _CHUNK // _L


def _sc_body(x_hbm, w_hbm, out_hbm, x0, x1, o0, o1, w_v, si0, si1, so0, so1):
    wid = lax.axis_index("s") * _NC + lax.axis_index("c")
    c0 = wid * _COLS_W

    pltpu.sync_copy(w_hbm, w_v)
    w = w_v[...]
    e = jnp.exp(w + w)
    lut = (e - 1.0) / (e + 1.0)

    xb, ob = (x0, x1), (o0, o1)
    sin, sout = (si0, si1), (so0, so1)

    def compute(x_v, o_v):
        def row_step(r, carry):
            for v in range(_VPC):
                idx = x_v[r, pl.ds(v * _L, _L)]
                o_v[r, pl.ds(v * _L, _L)] = lut.at[idx].get(
                    mode="promise_in_bounds")
            return carry

        lax.fori_loop(0, _F, row_step, 0)

    in_d = [None] * _NCH
    out_d = [None] * _NCH
    in_d[0] = pltpu.async_copy(x_hbm.at[:, pl.ds(c0, _CHUNK)], x0, si0)
    in_d[1] = pltpu.async_copy(x_hbm.at[:, pl.ds(c0 + _CHUNK, _CHUNK)], x1, si1)
    for k in range(_NCH):
        s = k & 1
        in_d[k].wait()
        if k >= 2:
            out_d[k - 2].wait()
        compute(xb[s], ob[s])
        out_d[k] = pltpu.async_copy(
            ob[s], out_hbm.at[:, pl.ds(c0 + k * _CHUNK, _CHUNK)], sout[s])
        if k + 2 < _NCH:
            in_d[k + 2] = pltpu.async_copy(
                x_hbm.at[:, pl.ds(c0 + (k + 2) * _CHUNK, _CHUNK)], xb[s], sin[s])
    out_d[_NCH - 2].wait()
    out_d[_NCH - 1].wait()


@functools.partial(jax.jit, static_argnames=())
def _run(xt, w_pad):
    mesh = plsc.VectorSubcoreMesh(core_axis_name="c", subcore_axis_name="s")
    f = pl.kernel(
        _sc_body,
        out_type=jax.ShapeDtypeStruct((_F, _B), jnp.float32),
        mesh=mesh,
        scratch_types=[
            pltpu.VMEM((_F, _CHUNK), jnp.int32),
            pltpu.VMEM((_F, _CHUNK), jnp.int32),
            pltpu.VMEM((_F, _CHUNK), jnp.float32),
            pltpu.VMEM((_F, _CHUNK), jnp.float32),
            pltpu.VMEM((_L,), jnp.float32),
            pltpu.SemaphoreType.DMA,
            pltpu.SemaphoreType.DMA,
            pltpu.SemaphoreType.DMA,
            pltpu.SemaphoreType.DMA,
        ],
    )
    return f(xt, w_pad)


def kernel(x, data_bias_weight):
    xt = x.T
    w_pad = jnp.pad(data_bias_weight.reshape(3), (0, _L - 3))
    yt = _run(xt, w_pad)
    return yt.T.reshape(_B, _F, 1)
